# baseline (speedup 1.0000x reference)
import functools

import jax
import jax.numpy as jnp
from jax import lax
from jax.experimental import pallas as pl
from jax.experimental.pallas import tpu as pltpu
from jax.experimental.pallas import tpu_sc as plsc

N = 2
HW = 4097
D = 768
K = 1024
L = 16
NC = 2
NS = 16
CH_PER_W = (N * D) // (NC * NS)
NSLAB = CH_PER_W // L
NB = 256
CH = 1024
NCH = (HW - 1) // CH


def _topk_body(xt_hbm, out_hbm, cbufA, cbufB, tslab, hist, slivin, slivout,
               semA, semB, semOA, semOB):
    wid = lax.axis_index("s") * NC + lax.axis_index("c")
    n = wid // (D // CH_PER_W)
    cbase = (wid % (D // CH_PER_W)) * CH_PER_W

    lane = lax.iota(jnp.int32, L)
    lane16 = lane * 16
    ones = jnp.ones((L,), jnp.int32)
    zeros = jnp.zeros((L,), jnp.int32)
    zf = jnp.zeros((L,), jnp.float32)
    kfull = jnp.full((L,), K, jnp.int32)

    bufs = (cbufA, cbufB)
    isems = (semA, semB)
    osems = (semOA, semOB)

    @plsc.parallel_loop(0, NB, unroll=8)
    def _(b):
        hist[pl.ds(b * L, L)] = zeros

    pltpu.sync_copy(xt_hbm.at[n, pl.ds(cbase, CH_PER_W), pl.ds(HW - 1, 1)],
                    slivin)

    def slab_body(s, _):
        c0 = cbase + s * L

        handles = {}
        handles[0] = pltpu.async_copy(
            xt_hbm.at[n, pl.ds(c0, L), pl.ds(0, CH)], cbufA, semA)
        for q in range(NCH):
            if q + 1 < NCH:
                handles[q + 1] = pltpu.async_copy(
                    xt_hbm.at[n, pl.ds(c0, L), pl.ds((q + 1) * CH, CH)],
                    bufs[(q + 1) % 2], isems[(q + 1) % 2])
            handles[q].wait()
            buf = bufs[q % 2]
            base0 = q * CH * L
            for c in range(L):
                skew_c = lane16 + ((lane + c) & 15)

                @plsc.parallel_loop(0, CH // L, unroll=8)
                def _(j, c=c, skew_c=skew_c, base0=base0, buf=buf):
                    v = buf[c, pl.ds(j * L, L)]
                    idx = skew_c + (base0 + j * L * L)
                    plsc.store_scatter(tslab, [idx], v)

        vlast = plsc.load_gather(slivin, [s * L + lane, zeros])
        plsc.store_scatter(tslab, [lane + (HW - 1) * L], vlast)

        kk = kfull
        t_prefix = zeros
        for p in range(3):
            shift = 23 - 8 * p
            cvecs = [(lane - m) & 15 for m in range(L)]
            if p == 0:
                tps = [None] * L
            else:
                tps = [t_prefix.at[cv].get(mode="promise_in_bounds")
                       for cv in cvecs]

            def hrow(off, m, shift=shift, tps=tps, cvecs=cvecs, p=p):
                bits = plsc.bitcast(tslab[pl.ds(off, L)],
                                    jnp.int32) & 0x7FFFFFFF
                b = (bits >> shift) & 0xFF
                if p == 0:
                    plsc.addupdate_scatter(hist, [b * L + cvecs[m]], ones)
                else:
                    msk = (bits >> (shift + 8)) == tps[m]
                    plsc.addupdate_scatter(hist, [b * L + cvecs[m]], ones,
                                           mask=msk)

            for m in range(1, L):
                hrow(m * L, m)

            @plsc.parallel_loop(1, NB, unroll=2)
            def _(blk):
                for m in range(L):
                    hrow(blk * NB + m * L, m)

            hrow((HW - 1) * L, 0)

            @plsc.parallel_loop(0, NB, unroll=4, carry=(zeros, zeros, zeros))
            def scan_out(j, carry):
                acc, bsel, above = carry
                b = NB - 1 - j
                h = hist[pl.ds(b * L, L)]
                hist[pl.ds(b * L, L)] = zeros
                acc2 = acc + h
                crossed = (acc < kk) & (acc2 >= kk)
                bsel = jnp.where(crossed, zeros + b, bsel)
                above = jnp.where(crossed, acc, above)
                return (acc2, bsel, above)

            _, bsel, above = scan_out
            kk = kk - above
            t_prefix = (t_prefix << 8) | bsel

        ohandles = {}
        for q in range(NCH):
            buf = bufs[q % 2]
            if q >= 2:
                ohandles[q - 2].wait()
            base0 = q * CH * L
            for c in range(L):
                tvec = jnp.full((L,), 0, jnp.int32) + t_prefix[c]
                cvec = zeros + c
                skew_c = lane16 + ((lane + c) & 15)

                @plsc.parallel_loop(0, CH // L, unroll=8)
                def _(j, c=c, skew_c=skew_c, base0=base0, tvec=tvec,
                      cvec=cvec, buf=buf, q=q):
                    idx = skew_c + (base0 + j * L * L)
                    v = plsc.load_gather(tslab, [idx])
                    bits = plsc.bitcast(v, jnp.int32) & 0x7FFFFFFF
                    keep = (bits >> 7) >= tvec
                    if q == 0:
                        keep = keep | (idx == cvec)
                    buf[c, pl.ds(j * L, L)] = jnp.where(keep, v, zf)

            ohandles[q] = pltpu.async_copy(
                buf, out_hbm.at[n, pl.ds(c0, L), pl.ds(q * CH, CH)],
                osems[q % 2])
        ohandles[NCH - 2].wait()
        ohandles[NCH - 1].wait()

        vlast = plsc.load_gather(tslab, [lane + (HW - 1) * L])
        lbits = plsc.bitcast(vlast, jnp.int32) & 0x7FFFFFFF
        vmasked = jnp.where((lbits >> 7) >= t_prefix, vlast, zf)
        plsc.store_scatter(slivout, [s * L + lane, zeros], vmasked)
        return 0

    lax.fori_loop(0, NSLAB, slab_body, 0)
    pltpu.sync_copy(slivout,
                    out_hbm.at[n, pl.ds(cbase, CH_PER_W), pl.ds(HW - 1, 1)])


_topk_call = functools.partial(
    pl.kernel,
    out_type=jax.ShapeDtypeStruct((N, D, HW), jnp.float32),
    mesh=plsc.VectorSubcoreMesh(core_axis_name="c", subcore_axis_name="s"),
    scratch_types=[
        pltpu.VMEM((L, CH), jnp.float32),
        pltpu.VMEM((L, CH), jnp.float32),
        pltpu.VMEM((HW * L,), jnp.float32),
        pltpu.VMEM((NB * L,), jnp.int32),
        pltpu.VMEM((CH_PER_W, 1), jnp.float32),
        pltpu.VMEM((CH_PER_W, 1), jnp.float32),
        pltpu.SemaphoreType.DMA,
        pltpu.SemaphoreType.DMA,
        pltpu.SemaphoreType.DMA,
        pltpu.SemaphoreType.DMA,
    ],
    compiler_params=pltpu.CompilerParams(needs_layout_passes=False),
)(_topk_body)


@jax.jit
def kernel(x):
    xt = jnp.transpose(x, (0, 2, 1))
    yt = _topk_call(xt)
    return jnp.transpose(yt, (0, 2, 1))


# speedup vs baseline: 1.0499x; 1.0499x over previous
import functools

import jax
import jax.numpy as jnp
from jax import lax
from jax.experimental import pallas as pl
from jax.experimental.pallas import tpu as pltpu
from jax.experimental.pallas import tpu_sc as plsc

N = 2
HW = 4097
D = 768
K = 1024
L = 16
NC = 2
NS = 16
CH_PER_W = (N * D) // (NC * NS)
NSLAB = CH_PER_W // L
NB = 256
CH = 1024
NCH = (HW - 1) // CH


def _topk_body(xt_hbm, out_hbm, cbufA, cbufB, tslab, hist, slivin, slivout,
               semA, semB, semOA, semOB):
    wid = lax.axis_index("s") * NC + lax.axis_index("c")
    n = wid // (D // CH_PER_W)
    cbase = (wid % (D // CH_PER_W)) * CH_PER_W

    lane = lax.iota(jnp.int32, L)
    lane16 = lane * 16
    ones = jnp.ones((L,), jnp.int32)
    zeros = jnp.zeros((L,), jnp.int32)
    zf = jnp.zeros((L,), jnp.float32)
    kfull = jnp.full((L,), K, jnp.int32)

    bufs = (cbufA, cbufB)
    isems = (semA, semB)
    osems = (semOA, semOB)

    @plsc.parallel_loop(0, NB, unroll=8)
    def _(b):
        hist[pl.ds(b * L, L)] = zeros

    pltpu.sync_copy(xt_hbm.at[n, pl.ds(cbase, CH_PER_W), pl.ds(HW - 1, 1)],
                    slivin)

    def slab_body(s, _):
        c0 = cbase + s * L

        handles = {}
        handles[0] = pltpu.async_copy(
            xt_hbm.at[n, pl.ds(c0, L), pl.ds(0, CH)], cbufA, semA)
        for q in range(NCH):
            if q + 1 < NCH:
                handles[q + 1] = pltpu.async_copy(
                    xt_hbm.at[n, pl.ds(c0, L), pl.ds((q + 1) * CH, CH)],
                    bufs[(q + 1) % 2], isems[(q + 1) % 2])
            handles[q].wait()
            buf = bufs[q % 2]
            base0 = q * CH * L
            skews = [lane16 + ((lane + c) & 15) for c in range(L)]

            @plsc.parallel_loop(0, CH // L, unroll=1)
            def _(j, skews=skews, base0=base0, buf=buf):
                base = base0 + j * L * L
                for c in range(L):
                    v = buf[c, pl.ds(j * L, L)]
                    plsc.store_scatter(tslab, [skews[c] + base], v)

        vlast = plsc.load_gather(slivin, [s * L + lane, zeros])
        plsc.store_scatter(tslab, [lane + (HW - 1) * L], vlast)

        kk = kfull
        t_prefix = zeros
        for p in range(3):
            shift = 23 - 8 * p
            cvecs = [(lane - m) & 15 for m in range(L)]
            if p == 0:
                tps = [None] * L
            else:
                tps = [t_prefix.at[cv].get(mode="promise_in_bounds")
                       for cv in cvecs]

            def hrow(off, m, shift=shift, tps=tps, cvecs=cvecs, p=p):
                bits = plsc.bitcast(tslab[pl.ds(off, L)],
                                    jnp.int32) & 0x7FFFFFFF
                b = (bits >> shift) & 0xFF
                if p == 0:
                    plsc.addupdate_scatter(hist, [b * L + cvecs[m]], ones)
                else:
                    msk = (bits >> (shift + 8)) == tps[m]
                    plsc.addupdate_scatter(hist, [b * L + cvecs[m]], ones,
                                           mask=msk)

            for m in range(1, L):
                hrow(m * L, m)

            @plsc.parallel_loop(1, NB, unroll=1)
            def _(blk):
                for m in range(L):
                    hrow(blk * NB + m * L, m)

            hrow((HW - 1) * L, 0)

            @plsc.parallel_loop(0, NB, unroll=4, carry=(zeros, zeros, zeros))
            def scan_out(j, carry):
                acc, bsel, above = carry
                b = NB - 1 - j
                h = hist[pl.ds(b * L, L)]
                hist[pl.ds(b * L, L)] = zeros
                acc2 = acc + h
                crossed = (acc < kk) & (acc2 >= kk)
                bsel = jnp.where(crossed, zeros + b, bsel)
                above = jnp.where(crossed, acc, above)
                return (acc2, bsel, above)

            _, bsel, above = scan_out
            kk = kk - above
            t_prefix = (t_prefix << 8) | bsel

        ohandles = {}
        for q in range(NCH):
            buf = bufs[q % 2]
            if q >= 2:
                ohandles[q - 2].wait()
            base0 = q * CH * L
            tvecs = [jnp.full((L,), 0, jnp.int32) + t_prefix[c]
                     for c in range(L)]
            skews = [lane16 + ((lane + c) & 15) for c in range(L)]

            @plsc.parallel_loop(0, CH // L, unroll=1)
            def _(j, skews=skews, base0=base0, tvecs=tvecs, buf=buf, q=q):
                base = base0 + j * L * L
                for c in range(L):
                    idx = skews[c] + base
                    v = plsc.load_gather(tslab, [idx])
                    bits = plsc.bitcast(v, jnp.int32) & 0x7FFFFFFF
                    keep = (bits >> 7) >= tvecs[c]
                    if q == 0:
                        keep = keep | (idx == (zeros + c))
                    buf[c, pl.ds(j * L, L)] = jnp.where(keep, v, zf)

            ohandles[q] = pltpu.async_copy(
                buf, out_hbm.at[n, pl.ds(c0, L), pl.ds(q * CH, CH)],
                osems[q % 2])
        ohandles[NCH - 2].wait()
        ohandles[NCH - 1].wait()

        vlast = plsc.load_gather(tslab, [lane + (HW - 1) * L])
        lbits = plsc.bitcast(vlast, jnp.int32) & 0x7FFFFFFF
        vmasked = jnp.where((lbits >> 7) >= t_prefix, vlast, zf)
        plsc.store_scatter(slivout, [s * L + lane, zeros], vmasked)
        return 0

    lax.fori_loop(0, NSLAB, slab_body, 0)
    pltpu.sync_copy(slivout,
                    out_hbm.at[n, pl.ds(cbase, CH_PER_W), pl.ds(HW - 1, 1)])


_topk_call = functools.partial(
    pl.kernel,
    out_type=jax.ShapeDtypeStruct((N, D, HW), jnp.float32),
    mesh=plsc.VectorSubcoreMesh(core_axis_name="c", subcore_axis_name="s"),
    scratch_types=[
        pltpu.VMEM((L, CH), jnp.float32),
        pltpu.VMEM((L, CH), jnp.float32),
        pltpu.VMEM((HW * L,), jnp.float32),
        pltpu.VMEM((NB * L,), jnp.int32),
        pltpu.VMEM((CH_PER_W, 1), jnp.float32),
        pltpu.VMEM((CH_PER_W, 1), jnp.float32),
        pltpu.SemaphoreType.DMA,
        pltpu.SemaphoreType.DMA,
        pltpu.SemaphoreType.DMA,
        pltpu.SemaphoreType.DMA,
    ],
    compiler_params=pltpu.CompilerParams(needs_layout_passes=False),
)(_topk_body)


@jax.jit
def kernel(x):
    xt = jnp.transpose(x, (0, 2, 1))
    yt = _topk_call(xt)
    return jnp.transpose(yt, (0, 2, 1))
